# baseline (speedup 1.0000x reference)
import jax
import jax.numpy as jnp
from jax.experimental import pallas as pl
from jax.experimental.pallas import tpu as pltpu
from jax.experimental.pallas import tpu_sc as plsc

BATCH = 16384
NUM_CLASSES = 1000
NUM_WORKERS = 32
ROWS_PER_WORKER = BATCH // NUM_WORKERS
CHUNK_ROWS = 32
CHUNKS_PER_WORKER = ROWS_PER_WORKER // CHUNK_ROWS
COL_TILES = [(t * 128, 128) for t in range(7)] + [(896, 104)]

_mesh = plsc.VectorSubcoreMesh(
    core_axis_name="c", subcore_axis_name="s", num_cores=2
)


@pl.kernel(
    mesh=_mesh,
    out_type=jax.ShapeDtypeStruct((BATCH, NUM_CLASSES), jnp.float32),
    scratch_types=[
        pltpu.VMEM((ROWS_PER_WORKER,), jnp.int32),
        pltpu.VMEM((CHUNK_ROWS, NUM_CLASSES), jnp.float32),
        pltpu.VMEM((CHUNK_ROWS, NUM_CLASSES), jnp.float32),
        pltpu.SemaphoreType.DMA,
        pltpu.SemaphoreType.DMA,
    ],
    compiler_params=pltpu.CompilerParams(needs_layout_passes=False),
)
def _onehot_sc(x_hbm, zblk_hbm, out_hbm, xs, buf_a, buf_b, sem_a, sem_b):
    wid = jax.lax.axis_index("s") * 2 + jax.lax.axis_index("c")
    base = wid * ROWS_PER_WORKER
    pltpu.sync_copy(x_hbm.at[pl.ds(base, ROWS_PER_WORKER)], xs)
    pltpu.sync_copy(zblk_hbm, buf_a)
    pltpu.sync_copy(zblk_hbm, buf_b)

    lane = jax.lax.iota(jnp.int32, 16)
    ones = jnp.full((16,), 1.0, jnp.float32)
    zeros = jnp.full((16,), 0.0, jnp.float32)

    def _scatter(buf, ci, vals):
        for half in range(2):
            cols = xs[pl.ds(ci * CHUNK_ROWS + half * 16, 16)]
            rows = lane + half * 16
            plsc.store_scatter(buf, [rows, cols], vals)

    def _tile_copies(buf, sem, row0):
        for g in range(CHUNK_ROWS // 8):
            for c0, w in COL_TILES:
                yield pltpu.make_async_copy(
                    buf.at[pl.ds(g * 8, 8), pl.ds(c0, w)],
                    out_hbm.at[pl.ds(row0 + g * 8, 8), pl.ds(c0, w)],
                    sem,
                )

    def _step(buf, sem, ci, k):
        @pl.when(k >= 1)
        def _drain():
            for cp in _tile_copies(buf, sem, 0):
                cp.wait()
            _scatter(buf, ci - 2, zeros)

        _scatter(buf, ci, ones)
        for cp in _tile_copies(buf, sem, base + ci * CHUNK_ROWS):
            cp.start()

    def _body(k, carry):
        _step(buf_a, sem_a, 2 * k, k)
        _step(buf_b, sem_b, 2 * k + 1, k)
        return carry

    jax.lax.fori_loop(0, CHUNKS_PER_WORKER // 2, _body, 0)
    for cp in _tile_copies(buf_a, sem_a, 0):
        cp.wait()
    for cp in _tile_copies(buf_b, sem_b, 0):
        cp.wait()


def kernel(x):
    xf = x.astype(jnp.int32).reshape(BATCH)
    zblk = jnp.zeros((CHUNK_ROWS, NUM_CLASSES), jnp.float32)
    return _onehot_sc(xf, zblk)


# speedup vs baseline: 1.0212x; 1.0212x over previous
import jax
import jax.numpy as jnp
from jax.experimental import pallas as pl
from jax.experimental.pallas import tpu as pltpu
from jax.experimental.pallas import tpu_sc as plsc

BATCH = 16384
NUM_CLASSES = 1000
NUM_WORKERS = 32
ROWS_PER_WORKER = BATCH // NUM_WORKERS
CHUNK_ROWS = 32
CHUNKS_PER_WORKER = ROWS_PER_WORKER // CHUNK_ROWS

_mesh = plsc.VectorSubcoreMesh(
    core_axis_name="c", subcore_axis_name="s", num_cores=2
)


@pl.kernel(
    mesh=_mesh,
    out_type=jax.ShapeDtypeStruct((BATCH, NUM_CLASSES), jnp.float32),
    scratch_types=[
        pltpu.VMEM((ROWS_PER_WORKER,), jnp.int32),
        pltpu.VMEM((CHUNK_ROWS, NUM_CLASSES), jnp.float32),
        pltpu.VMEM((CHUNK_ROWS, NUM_CLASSES), jnp.float32),
        pltpu.SemaphoreType.DMA,
        pltpu.SemaphoreType.DMA,
    ],
    compiler_params=pltpu.CompilerParams(needs_layout_passes=False),
)
def _onehot_sc(x_hbm, zblk_hbm, out_hbm, xs, buf_a, buf_b, sem_a, sem_b):
    wid = jax.lax.axis_index("s") * 2 + jax.lax.axis_index("c")
    base = wid * ROWS_PER_WORKER
    pltpu.sync_copy(x_hbm.at[pl.ds(base, ROWS_PER_WORKER)], xs)
    pltpu.sync_copy(zblk_hbm, buf_a)
    pltpu.sync_copy(zblk_hbm, buf_b)

    lane = jax.lax.iota(jnp.int32, 16)
    ones = jnp.full((16,), 1.0, jnp.float32)
    zeros = jnp.full((16,), 0.0, jnp.float32)

    def _scatter(buf, ci, vals):
        for half in range(2):
            cols = xs[pl.ds(ci * CHUNK_ROWS + half * 16, 16)]
            rows = lane + half * 16
            plsc.store_scatter(buf, [rows, cols], vals)

    def _step(buf, sem, ci, k):
        @pl.when(k >= 1)
        def _drain():
            pltpu.make_async_copy(
                buf, out_hbm.at[pl.ds(0, CHUNK_ROWS)], sem
            ).wait()
            _scatter(buf, ci - 2, zeros)

        _scatter(buf, ci, ones)
        pltpu.make_async_copy(
            buf, out_hbm.at[pl.ds(base + ci * CHUNK_ROWS, CHUNK_ROWS)], sem
        ).start()

    def _body(k, carry):
        _step(buf_a, sem_a, 2 * k, k)
        _step(buf_b, sem_b, 2 * k + 1, k)
        return carry

    jax.lax.fori_loop(0, CHUNKS_PER_WORKER // 2, _body, 0)
    pltpu.make_async_copy(buf_a, out_hbm.at[pl.ds(0, CHUNK_ROWS)], sem_a).wait()
    pltpu.make_async_copy(buf_b, out_hbm.at[pl.ds(0, CHUNK_ROWS)], sem_b).wait()


def kernel(x):
    xf = x.astype(jnp.int32).reshape(BATCH)
    zblk = jnp.zeros((CHUNK_ROWS, NUM_CLASSES), jnp.float32)
    return _onehot_sc(xf, zblk)
